# baseline (speedup 1.0000x reference)
import functools

import jax
import jax.numpy as jnp
from jax import lax
from jax.experimental import pallas as pl
from jax.experimental.pallas import tpu as pltpu
from jax.experimental.pallas import tpu_sc as plsc

T = 2048
D = 768
E = 64
K = 2
F = 512

TM = 64
N_TILES = (T * K) // TM + (E - 1)
NG = N_TILES * TM

NW = 32
GB = (T * K) // NW
CB = T // NW
LANES = 16
DC = D // LANES


def _router_kernel(hs_ref, wg_ref, idx_ref, w_ref):
    logits = jnp.dot(hs_ref[...], wg_ref[...], preferred_element_type=jnp.float32)
    m = jnp.max(logits, axis=1, keepdims=True)
    p = jnp.exp(logits - m)
    p = p / jnp.sum(p, axis=1, keepdims=True)
    iota = jax.lax.broadcasted_iota(jnp.int32, (T, E), 1)
    m1 = jnp.max(p, axis=1, keepdims=True)
    i1 = jnp.min(jnp.where(p == m1, iota, E), axis=1, keepdims=True)
    p2 = jnp.where(iota == i1, -1e30, p)
    m2 = jnp.max(p2, axis=1, keepdims=True)
    i2 = jnp.min(jnp.where(p2 == m2, iota, E), axis=1, keepdims=True)
    s = m1 + m2
    idx_ref[...] = jnp.concatenate([i1, i2], axis=1)
    w_ref[...] = jnp.concatenate([m1 / s, m2 / s], axis=1)


def _ffn_kernel(te_ref, tx_ref, ty_ref, x_ref, wgu_ref, wd_ref, y_ref):
    gu = jnp.dot(x_ref[...], wgu_ref[0], preferred_element_type=jnp.float32)
    g = gu[:, :F]
    u = gu[:, F:]
    h = g * jax.nn.sigmoid(g) * u
    y_ref[...] = jnp.dot(h, wd_ref[0], preferred_element_type=jnp.float32)


_sc_mesh = plsc.VectorSubcoreMesh(core_axis_name="c", subcore_axis_name="s")


@functools.partial(
    pl.kernel,
    mesh=_sc_mesh,
    out_type=jax.ShapeDtypeStruct((NG, D), jnp.float32),
    scratch_types=[
        pltpu.VMEM((GB,), jnp.int32),
        pltpu.VMEM((GB,), jnp.int32),
        pltpu.VMEM((GB, D), jnp.float32),
        pltpu.SemaphoreType.DMA,
        pltpu.SemaphoreType.DMA,
    ],
)
def _sc_dispatch(hs_hbm, tok_hbm, slot_hbm, xs_hbm, tok_v, slot_v, rows_v,
                 sem_g, sem_s):
    wid = lax.axis_index("s") * 2 + lax.axis_index("c")
    base = wid * GB
    pltpu.sync_copy(tok_hbm.at[pl.ds(base, GB)], tok_v)
    pltpu.sync_copy(slot_hbm.at[pl.ds(base, GB)], slot_v)
    pltpu.async_copy(hs_hbm.at[tok_v], rows_v, sem_g).wait()
    pltpu.async_copy(rows_v, xs_hbm.at[slot_v], sem_s).wait()


@functools.partial(
    pl.kernel,
    mesh=_sc_mesh,
    out_type=jax.ShapeDtypeStruct((T, D), jnp.float32),
    scratch_types=[
        pltpu.VMEM((CB,), jnp.int32),
        pltpu.VMEM((CB,), jnp.int32),
        pltpu.VMEM((CB, D), jnp.float32),
        pltpu.VMEM((CB, D), jnp.float32),
        pltpu.VMEM((CB * LANES,), jnp.float32),
        pltpu.VMEM((CB * LANES,), jnp.float32),
        pltpu.SemaphoreType.DMA,
        pltpu.SemaphoreType.DMA,
    ],
)
def _sc_combine(ys_hbm, sa_hbm, sb_hbm, wa_hbm, wb_hbm, out_hbm,
                sa_v, sb_v, ya_v, yb_v, wa_v, wb_v, sem_a, sem_b):
    wid = lax.axis_index("s") * 2 + lax.axis_index("c")
    base = wid * CB
    pltpu.sync_copy(sa_hbm.at[pl.ds(base, CB)], sa_v)
    pltpu.sync_copy(sb_hbm.at[pl.ds(base, CB)], sb_v)
    pltpu.sync_copy(wa_hbm.at[pl.ds(base * LANES, CB * LANES)], wa_v)
    pltpu.sync_copy(wb_hbm.at[pl.ds(base * LANES, CB * LANES)], wb_v)
    ga = pltpu.async_copy(ys_hbm.at[sa_v], ya_v, sem_a)
    gb = pltpu.async_copy(ys_hbm.at[sb_v], yb_v, sem_b)
    ga.wait()
    gb.wait()

    def row_body(r, carry):
        wa = wa_v[pl.ds(r * LANES, LANES)]
        wb = wb_v[pl.ds(r * LANES, LANES)]
        for c in range(DC):
            ya = ya_v[r, pl.ds(c * LANES, LANES)]
            yb = yb_v[r, pl.ds(c * LANES, LANES)]
            ya_v[r, pl.ds(c * LANES, LANES)] = wa * ya + wb * yb
        return carry

    lax.fori_loop(0, CB, row_body, 0)
    pltpu.sync_copy(ya_v, out_hbm.at[pl.ds(base, CB)])


def kernel(hidden_states, W_gate, W_gate_up, W_down, num_global_tokens,
           max_num_tokens_per_gpu):
    hs = hidden_states
    topk_idx, topk_w = pl.pallas_call(
        _router_kernel,
        out_shape=(
            jax.ShapeDtypeStruct((T, K), jnp.int32),
            jax.ShapeDtypeStruct((T, K), jnp.float32),
        ),
    )(hs, W_gate)

    e_flat = topk_idx.reshape(-1)
    t_flat = jnp.repeat(jnp.arange(T, dtype=jnp.int32), K)
    onehot = e_flat[:, None] == jnp.arange(E, dtype=jnp.int32)[None, :]
    csum = jnp.cumsum(onehot.astype(jnp.int32), axis=0)
    counts = csum[-1]
    rank = jnp.sum(jnp.where(onehot, csum - 1, 0), axis=1)
    padded = ((counts + TM - 1) // TM) * TM
    pad_end = jnp.cumsum(padded)
    pad_start = pad_end - padded
    slot = (pad_start[e_flat] + rank).astype(jnp.int32)
    total = pad_end[-1]
    tile_start = jnp.arange(N_TILES, dtype=jnp.int32) * TM
    te = jnp.searchsorted(pad_end, jnp.minimum(tile_start, total - 1),
                          side="right").astype(jnp.int32)
    tile_expert = jnp.minimum(te, E - 1)
    live = tile_start < total
    tile_ids = jnp.arange(N_TILES, dtype=jnp.int32)
    tile_x = jnp.where(live, tile_ids, 0)
    tile_y = jnp.where(live, tile_ids, N_TILES)

    x_sorted = _sc_dispatch(hs, t_flat, slot)

    grid_spec = pltpu.PrefetchScalarGridSpec(
        num_scalar_prefetch=3,
        grid=(N_TILES,),
        in_specs=[
            pl.BlockSpec((TM, D), lambda i, te, tx, ty: (tx[i], 0)),
            pl.BlockSpec((1, D, 2 * F), lambda i, te, tx, ty: (te[i], 0, 0)),
            pl.BlockSpec((1, F, D), lambda i, te, tx, ty: (te[i], 0, 0)),
        ],
        out_specs=pl.BlockSpec((TM, D), lambda i, te, tx, ty: (ty[i], 0)),
    )
    y_sorted = pl.pallas_call(
        _ffn_kernel,
        grid_spec=grid_spec,
        out_shape=jax.ShapeDtypeStruct(((N_TILES + 1) * TM, D), jnp.float32),
    )(tile_expert, tile_x, tile_y, x_sorted, W_gate_up, W_down)

    inv_slot = slot.reshape(T, K)
    wa_b = jnp.broadcast_to(topk_w[:, 0:1], (T, LANES)).reshape(T * LANES)
    wb_b = jnp.broadcast_to(topk_w[:, 1:2], (T, LANES)).reshape(T * LANES)
    out = _sc_combine(y_sorted, inv_slot[:, 0], inv_slot[:, 1], wa_b, wb_b)
    return out


# speedup vs baseline: 1.1405x; 1.1405x over previous
import functools

import jax
import jax.numpy as jnp
from jax import lax
from jax.experimental import pallas as pl
from jax.experimental.pallas import tpu as pltpu
from jax.experimental.pallas import tpu_sc as plsc

T = 2048
D = 768
E = 64
K = 2
F = 512

TM = 128
N_TILES = (T * K) // TM + (E - 1)
NG = N_TILES * TM

NW = 32
GB = (T * K) // NW
CB = T // NW
LANES = 16
DC = D // LANES


def _router_kernel(hs_ref, wg_ref, idx_ref, w_ref):
    logits = jnp.dot(hs_ref[...], wg_ref[...], preferred_element_type=jnp.float32)
    m = jnp.max(logits, axis=1, keepdims=True)
    p = jnp.exp(logits - m)
    p = p / jnp.sum(p, axis=1, keepdims=True)
    iota = jax.lax.broadcasted_iota(jnp.int32, (T, E), 1)
    m1 = jnp.max(p, axis=1, keepdims=True)
    i1 = jnp.min(jnp.where(p == m1, iota, E), axis=1, keepdims=True)
    p2 = jnp.where(iota == i1, -1e30, p)
    m2 = jnp.max(p2, axis=1, keepdims=True)
    i2 = jnp.min(jnp.where(p2 == m2, iota, E), axis=1, keepdims=True)
    s = m1 + m2
    idx_ref[...] = jnp.concatenate([i1, i2], axis=1)
    w_ref[...] = jnp.concatenate([m1 / s, m2 / s], axis=1)


def _ffn_kernel(te_ref, tx_ref, ty_ref, x_ref, wgu_ref, wd_ref, y_ref):
    gu = jnp.dot(x_ref[...], wgu_ref[0], preferred_element_type=jnp.float32)
    g = gu[:, :F]
    u = gu[:, F:]
    h = g * jax.nn.sigmoid(g) * u
    y_ref[...] = jnp.dot(h, wd_ref[0], preferred_element_type=jnp.float32)


_sc_mesh = plsc.VectorSubcoreMesh(core_axis_name="c", subcore_axis_name="s")


@functools.partial(
    pl.kernel,
    mesh=_sc_mesh,
    out_type=jax.ShapeDtypeStruct((NG, D), jnp.float32),
    scratch_types=[
        pltpu.VMEM((GB,), jnp.int32),
        pltpu.VMEM((GB,), jnp.int32),
        pltpu.VMEM((GB, D), jnp.float32),
        pltpu.SemaphoreType.DMA,
        pltpu.SemaphoreType.DMA,
    ],
)
def _sc_dispatch(hs_hbm, tok_hbm, slot_hbm, xs_hbm, tok_v, slot_v, rows_v,
                 sem_g, sem_s):
    wid = lax.axis_index("s") * 2 + lax.axis_index("c")
    base = wid * GB
    pltpu.sync_copy(tok_hbm.at[pl.ds(base, GB)], tok_v)
    pltpu.sync_copy(slot_hbm.at[pl.ds(base, GB)], slot_v)
    pltpu.async_copy(hs_hbm.at[tok_v], rows_v, sem_g).wait()
    pltpu.async_copy(rows_v, xs_hbm.at[slot_v], sem_s).wait()


@functools.partial(
    pl.kernel,
    mesh=_sc_mesh,
    out_type=jax.ShapeDtypeStruct((T, D), jnp.float32),
    scratch_types=[
        pltpu.VMEM((CB,), jnp.int32),
        pltpu.VMEM((CB,), jnp.int32),
        pltpu.VMEM((CB, D), jnp.float32),
        pltpu.VMEM((CB, D), jnp.float32),
        pltpu.VMEM((CB * LANES,), jnp.float32),
        pltpu.VMEM((CB * LANES,), jnp.float32),
        pltpu.SemaphoreType.DMA,
        pltpu.SemaphoreType.DMA,
    ],
)
def _sc_combine(ys_hbm, sa_hbm, sb_hbm, wa_hbm, wb_hbm, out_hbm,
                sa_v, sb_v, ya_v, yb_v, wa_v, wb_v, sem_a, sem_b):
    wid = lax.axis_index("s") * 2 + lax.axis_index("c")
    base = wid * CB
    pltpu.sync_copy(sa_hbm.at[pl.ds(base, CB)], sa_v)
    pltpu.sync_copy(sb_hbm.at[pl.ds(base, CB)], sb_v)
    pltpu.sync_copy(wa_hbm.at[pl.ds(base * LANES, CB * LANES)], wa_v)
    pltpu.sync_copy(wb_hbm.at[pl.ds(base * LANES, CB * LANES)], wb_v)
    ga = pltpu.async_copy(ys_hbm.at[sa_v], ya_v, sem_a)
    gb = pltpu.async_copy(ys_hbm.at[sb_v], yb_v, sem_b)
    ga.wait()
    gb.wait()

    def row_body(r, carry):
        wa = wa_v[pl.ds(r * LANES, LANES)]
        wb = wb_v[pl.ds(r * LANES, LANES)]
        for c in range(DC):
            ya = ya_v[r, pl.ds(c * LANES, LANES)]
            yb = yb_v[r, pl.ds(c * LANES, LANES)]
            ya_v[r, pl.ds(c * LANES, LANES)] = wa * ya + wb * yb
        return carry

    lax.fori_loop(0, CB, row_body, 0)
    pltpu.sync_copy(ya_v, out_hbm.at[pl.ds(base, CB)])


def kernel(hidden_states, W_gate, W_gate_up, W_down, num_global_tokens,
           max_num_tokens_per_gpu):
    hs = hidden_states
    topk_idx, topk_w = pl.pallas_call(
        _router_kernel,
        out_shape=(
            jax.ShapeDtypeStruct((T, K), jnp.int32),
            jax.ShapeDtypeStruct((T, K), jnp.float32),
        ),
    )(hs, W_gate)

    e_flat = topk_idx.reshape(-1)
    t_flat = jnp.repeat(jnp.arange(T, dtype=jnp.int32), K)
    onehot = e_flat[:, None] == jnp.arange(E, dtype=jnp.int32)[None, :]
    csum = jnp.cumsum(onehot.astype(jnp.int32), axis=0)
    counts = csum[-1]
    rank = jnp.sum(jnp.where(onehot, csum - 1, 0), axis=1)
    padded = ((counts + TM - 1) // TM) * TM
    pad_end = jnp.cumsum(padded)
    pad_start = pad_end - padded
    slot = (pad_start[e_flat] + rank).astype(jnp.int32)
    total = pad_end[-1]
    tile_start = jnp.arange(N_TILES, dtype=jnp.int32) * TM
    te = jnp.searchsorted(pad_end, jnp.minimum(tile_start, total - 1),
                          side="right").astype(jnp.int32)
    tile_expert = jnp.minimum(te, E - 1)
    live = tile_start < total
    tile_ids = jnp.arange(N_TILES, dtype=jnp.int32)
    tile_x = jnp.where(live, tile_ids, 0)
    tile_y = jnp.where(live, tile_ids, N_TILES)

    x_sorted = _sc_dispatch(hs, t_flat, slot)

    grid_spec = pltpu.PrefetchScalarGridSpec(
        num_scalar_prefetch=3,
        grid=(N_TILES,),
        in_specs=[
            pl.BlockSpec((TM, D), lambda i, te, tx, ty: (tx[i], 0)),
            pl.BlockSpec((1, D, 2 * F), lambda i, te, tx, ty: (te[i], 0, 0)),
            pl.BlockSpec((1, F, D), lambda i, te, tx, ty: (te[i], 0, 0)),
        ],
        out_specs=pl.BlockSpec((TM, D), lambda i, te, tx, ty: (ty[i], 0)),
    )
    y_sorted = pl.pallas_call(
        _ffn_kernel,
        grid_spec=grid_spec,
        out_shape=jax.ShapeDtypeStruct(((N_TILES + 1) * TM, D), jnp.float32),
    )(tile_expert, tile_x, tile_y, x_sorted, W_gate_up, W_down)

    inv_slot = slot.reshape(T, K)
    wa_b = jnp.broadcast_to(topk_w[:, 0:1], (T, LANES)).reshape(T * LANES)
    wb_b = jnp.broadcast_to(topk_w[:, 1:2], (T, LANES)).reshape(T * LANES)
    out = _sc_combine(y_sorted, inv_slot[:, 0], inv_slot[:, 1], wa_b, wb_b)
    return out


# speedup vs baseline: 1.5714x; 1.3778x over previous
import functools

import jax
import jax.numpy as jnp
from jax import lax
from jax.experimental import pallas as pl
from jax.experimental.pallas import tpu as pltpu
from jax.experimental.pallas import tpu_sc as plsc

T = 2048
D = 768
E = 64
K = 2
F = 512

TM = 128
N_TILES = (T * K) // TM + (E - 1)
NG = N_TILES * TM

NW = 32
GB = (T * K) // NW
CB = T // NW
LANES = 16
DC = D // LANES


BS = 256
NB = T // BS
NT_PAD = 128


def _router_kernel(hs_ref, wg_ref, w_ref, slots_ref, te_ref, tx_ref, ty_ref):
    logits = jnp.dot(hs_ref[...], wg_ref[...], preferred_element_type=jnp.float32)
    m = jnp.max(logits, axis=1, keepdims=True)
    p = jnp.exp(logits - m)
    p = p / jnp.sum(p, axis=1, keepdims=True)
    iota = jax.lax.broadcasted_iota(jnp.int32, (T, E), 1)
    m1 = jnp.max(p, axis=1, keepdims=True)
    i1 = jnp.min(jnp.where(p == m1, iota, E), axis=1, keepdims=True)
    p2 = jnp.where(iota == i1, -1e30, p)
    m2 = jnp.max(p2, axis=1, keepdims=True)
    i2 = jnp.min(jnp.where(p2 == m2, iota, E), axis=1, keepdims=True)
    s = m1 + m2
    w_ref[...] = jnp.concatenate([m1 / s, m2 / s], axis=1)

    oh1 = (iota == i1).astype(jnp.float32)
    oh2 = (iota == i2).astype(jnp.float32)
    cmat = oh1 + oh2
    r_i = jax.lax.broadcasted_iota(jnp.int32, (BS, BS), 0)
    c_i = jax.lax.broadcasted_iota(jnp.int32, (BS, BS), 1)
    lts = (c_i < r_i).astype(jnp.float32)
    counts = jnp.zeros((1, E), jnp.float32)
    blocks = []
    for b in range(NB):
        cb = cmat[b * BS:(b + 1) * BS, :]
        blocks.append(jnp.dot(lts, cb, preferred_element_type=jnp.float32)
                      + counts)
        counts = counts + jnp.sum(cb, axis=0, keepdims=True)
    cum = jnp.concatenate(blocks, axis=0)

    padded = jnp.floor((counts + (TM - 1)) * (1.0 / TM)) * TM
    ue = (jax.lax.broadcasted_iota(jnp.int32, (E, E), 0)
          <= jax.lax.broadcasted_iota(jnp.int32, (E, E), 1)).astype(jnp.float32)
    pad_end = jnp.dot(padded, ue, preferred_element_type=jnp.float32)
    pad_start = pad_end - padded
    total = jnp.sum(padded)

    slot1 = jnp.sum(oh1 * (pad_start + cum), axis=1, keepdims=True)
    slot2 = jnp.sum(oh2 * (pad_start + cum), axis=1, keepdims=True)
    slots_ref[...] = jnp.concatenate([slot1, slot2], axis=1).astype(jnp.int32)

    tcol = jax.lax.broadcasted_iota(jnp.int32, (NT_PAD, E), 0) * TM
    tscol = jnp.minimum(tcol.astype(jnp.float32), total - 1.0)
    cmp = (jnp.broadcast_to(pad_end, (NT_PAD, E)) <= tscol).astype(jnp.float32)
    te = jnp.minimum(jnp.sum(cmp, axis=1, keepdims=True), E - 1)
    te_ref[...] = te.astype(jnp.int32)
    live = tcol[:, :1].astype(jnp.float32) < total
    tid = jax.lax.broadcasted_iota(jnp.int32, (NT_PAD, 1), 0)
    tx_ref[...] = jnp.where(live, tid, 0)
    ty_ref[...] = jnp.where(live, tid, N_TILES)


def _ffn_kernel(te_ref, tx_ref, ty_ref, x_ref, wgu_ref, wd_ref, y_ref):
    gu = jnp.dot(x_ref[...], wgu_ref[0], preferred_element_type=jnp.float32)
    g = gu[:, :F]
    u = gu[:, F:]
    h = g * jax.nn.sigmoid(g) * u
    y_ref[...] = jnp.dot(h, wd_ref[0], preferred_element_type=jnp.float32)


def _sc_dispatch_body(hs_hbm, tok_hbm, slot_hbm, xs_hbm, tok_v, slot_v, rows_v,
                      sem_g, sem_s):
    wid = lax.axis_index("s") * 2 + lax.axis_index("c")
    base = wid * GB
    pltpu.sync_copy(tok_hbm.at[pl.ds(base, GB)], tok_v)
    pltpu.sync_copy(slot_hbm.at[pl.ds(base, GB)], slot_v)
    pltpu.async_copy(hs_hbm.at[tok_v], rows_v, sem_g).wait()
    pltpu.async_copy(rows_v, xs_hbm.at[slot_v], sem_s).wait()


def _sc_combine_body(ys_hbm, sa_hbm, sb_hbm, wa_hbm, wb_hbm, out_hbm,
                     sa_v, sb_v, ya_v, yb_v, wa_v, wb_v, sem_a, sem_b):
    wid = lax.axis_index("s") * 2 + lax.axis_index("c")
    base = wid * CB
    pltpu.sync_copy(sa_hbm.at[pl.ds(base, CB)], sa_v)
    pltpu.sync_copy(sb_hbm.at[pl.ds(base, CB)], sb_v)
    pltpu.sync_copy(wa_hbm.at[pl.ds(base * LANES, CB * LANES)], wa_v)
    pltpu.sync_copy(wb_hbm.at[pl.ds(base * LANES, CB * LANES)], wb_v)
    ga = pltpu.async_copy(ys_hbm.at[sa_v], ya_v, sem_a)
    gb = pltpu.async_copy(ys_hbm.at[sb_v], yb_v, sem_b)
    ga.wait()
    gb.wait()

    def row_body(r, carry):
        wa = wa_v[pl.ds(r * LANES, LANES)]
        wb = wb_v[pl.ds(r * LANES, LANES)]
        for c in range(DC):
            ya = ya_v[r, pl.ds(c * LANES, LANES)]
            yb = yb_v[r, pl.ds(c * LANES, LANES)]
            ya_v[r, pl.ds(c * LANES, LANES)] = wa * ya + wb * yb
        return carry

    lax.fori_loop(0, CB, row_body, 0)
    pltpu.sync_copy(ya_v, out_hbm.at[pl.ds(base, CB)])


def kernel(hidden_states, W_gate, W_gate_up, W_down, num_global_tokens,
           max_num_tokens_per_gpu):
    hs = hidden_states
    topk_w, slots, te2, tx2, ty2 = pl.pallas_call(
        _router_kernel,
        out_shape=(
            jax.ShapeDtypeStruct((T, K), jnp.float32),
            jax.ShapeDtypeStruct((T, K), jnp.int32),
            jax.ShapeDtypeStruct((NT_PAD, 1), jnp.int32),
            jax.ShapeDtypeStruct((NT_PAD, 1), jnp.int32),
            jax.ShapeDtypeStruct((NT_PAD, 1), jnp.int32),
        ),
    )(hs, W_gate)
    tile_expert = te2.reshape(-1)
    tile_x = tx2.reshape(-1)
    tile_y = ty2.reshape(-1)
    t_flat = jnp.repeat(jnp.arange(T, dtype=jnp.int32), K)
    slot = slots.reshape(-1)

    sc_mesh = plsc.VectorSubcoreMesh(core_axis_name="c", subcore_axis_name="s")
    sc_dispatch = pl.kernel(
        _sc_dispatch_body,
        mesh=sc_mesh,
        out_type=jax.ShapeDtypeStruct((NG, D), jnp.float32),
        scratch_types=[
            pltpu.VMEM((GB,), jnp.int32),
            pltpu.VMEM((GB,), jnp.int32),
            pltpu.VMEM((GB, D), jnp.float32),
            pltpu.SemaphoreType.DMA,
            pltpu.SemaphoreType.DMA,
        ],
    )
    x_sorted = sc_dispatch(hs, t_flat, slot)

    grid_spec = pltpu.PrefetchScalarGridSpec(
        num_scalar_prefetch=3,
        grid=(N_TILES,),
        in_specs=[
            pl.BlockSpec((TM, D), lambda i, te, tx, ty: (tx[i], 0)),
            pl.BlockSpec((1, D, 2 * F), lambda i, te, tx, ty: (te[i], 0, 0)),
            pl.BlockSpec((1, F, D), lambda i, te, tx, ty: (te[i], 0, 0)),
        ],
        out_specs=pl.BlockSpec((TM, D), lambda i, te, tx, ty: (ty[i], 0)),
    )
    y_sorted = pl.pallas_call(
        _ffn_kernel,
        grid_spec=grid_spec,
        out_shape=jax.ShapeDtypeStruct(((N_TILES + 1) * TM, D), jnp.float32),
    )(tile_expert, tile_x, tile_y, x_sorted, W_gate_up, W_down)

    wa_b = jnp.broadcast_to(topk_w[:, 0:1], (T, LANES)).reshape(T * LANES)
    wb_b = jnp.broadcast_to(topk_w[:, 1:2], (T, LANES)).reshape(T * LANES)
    sc_combine = pl.kernel(
        _sc_combine_body,
        mesh=sc_mesh,
        out_type=jax.ShapeDtypeStruct((T, D), jnp.float32),
        scratch_types=[
            pltpu.VMEM((CB,), jnp.int32),
            pltpu.VMEM((CB,), jnp.int32),
            pltpu.VMEM((CB, D), jnp.float32),
            pltpu.VMEM((CB, D), jnp.float32),
            pltpu.VMEM((CB * LANES,), jnp.float32),
            pltpu.VMEM((CB * LANES,), jnp.float32),
            pltpu.SemaphoreType.DMA,
            pltpu.SemaphoreType.DMA,
        ],
    )
    out = sc_combine(y_sorted, slots[:, 0], slots[:, 1], wa_b, wb_b)
    return out


# speedup vs baseline: 1.5958x; 1.0156x over previous
import functools

import jax
import jax.numpy as jnp
from jax import lax
from jax.experimental import pallas as pl
from jax.experimental.pallas import tpu as pltpu
from jax.experimental.pallas import tpu_sc as plsc

T = 2048
D = 768
E = 64
K = 2
F = 512

TM = 128
N_TILES = (T * K) // TM + (E - 1)
NG = N_TILES * TM

NW = 32
GB = (T * K) // NW
CB = T // NW
LANES = 16
DC = D // LANES


BS = 256
NB = T // BS
NT_PAD = 128


def _router_kernel(hs_ref, wg_ref, w_ref, slots_ref, te_ref, tx_ref, ty_ref):
    logits = jnp.dot(hs_ref[...], wg_ref[...], preferred_element_type=jnp.float32)
    m = jnp.max(logits, axis=1, keepdims=True)
    p = jnp.exp(logits - m)
    p = p / jnp.sum(p, axis=1, keepdims=True)
    iota = jax.lax.broadcasted_iota(jnp.int32, (T, E), 1)
    m1 = jnp.max(p, axis=1, keepdims=True)
    i1 = jnp.min(jnp.where(p == m1, iota, E), axis=1, keepdims=True)
    p2 = jnp.where(iota == i1, -1e30, p)
    m2 = jnp.max(p2, axis=1, keepdims=True)
    i2 = jnp.min(jnp.where(p2 == m2, iota, E), axis=1, keepdims=True)
    s = m1 + m2
    w_ref[...] = jnp.concatenate([m1 / s, m2 / s], axis=1)

    oh1 = (iota == i1).astype(jnp.float32)
    oh2 = (iota == i2).astype(jnp.float32)
    cmat = oh1 + oh2
    r_i = jax.lax.broadcasted_iota(jnp.int32, (BS, BS), 0)
    c_i = jax.lax.broadcasted_iota(jnp.int32, (BS, BS), 1)
    lts = (c_i < r_i).astype(jnp.float32)
    counts = jnp.zeros((1, E), jnp.float32)
    blocks = []
    for b in range(NB):
        cb = cmat[b * BS:(b + 1) * BS, :]
        blocks.append(jnp.dot(lts, cb, preferred_element_type=jnp.float32)
                      + counts)
        counts = counts + jnp.sum(cb, axis=0, keepdims=True)
    cum = jnp.concatenate(blocks, axis=0)

    padded = jnp.floor((counts + (TM - 1)) * (1.0 / TM)) * TM
    ue = (jax.lax.broadcasted_iota(jnp.int32, (E, E), 0)
          <= jax.lax.broadcasted_iota(jnp.int32, (E, E), 1)).astype(jnp.float32)
    pad_end = jnp.dot(padded, ue, preferred_element_type=jnp.float32)
    pad_start = pad_end - padded
    total = jnp.sum(padded)

    slot1 = jnp.sum(oh1 * (pad_start + cum), axis=1, keepdims=True)
    slot2 = jnp.sum(oh2 * (pad_start + cum), axis=1, keepdims=True)
    slots_ref[...] = jnp.concatenate([slot1, slot2], axis=1).astype(jnp.int32)

    tcol = jax.lax.broadcasted_iota(jnp.int32, (NT_PAD, E), 0) * TM
    tscol = jnp.minimum(tcol.astype(jnp.float32), total - 1.0)
    cmp = (jnp.broadcast_to(pad_end, (NT_PAD, E)) <= tscol).astype(jnp.float32)
    te = jnp.minimum(jnp.sum(cmp, axis=1, keepdims=True), E - 1)
    te_ref[...] = te.astype(jnp.int32)
    live = tcol[:, :1].astype(jnp.float32) < total
    tid = jax.lax.broadcasted_iota(jnp.int32, (NT_PAD, 1), 0)
    tx_ref[...] = jnp.where(live, tid, 0)
    ty_ref[...] = jnp.where(live, tid, N_TILES)


def _ffn_kernel(te_ref, tx_ref, ty_ref, x_ref, wgu_ref, wd_ref, y_ref):
    gu = jnp.dot(x_ref[...], wgu_ref[0], preferred_element_type=jnp.float32)
    g = gu[:, :F]
    u = gu[:, F:]
    h = g * jax.nn.sigmoid(g) * u
    y_ref[...] = jnp.dot(h, wd_ref[0], preferred_element_type=jnp.float32)


HG = GB // 2


def _sc_dispatch_body(hs_hbm, tok_hbm, slot_hbm, xs_hbm,
                      tok0_v, tok1_v, slot0_v, slot1_v, rows0_v, rows1_v,
                      sem_i, sem_g0, sem_g1, sem_s):
    wid = lax.axis_index("s") * 2 + lax.axis_index("c")
    base = wid * GB
    c0 = pltpu.async_copy(tok_hbm.at[pl.ds(base, HG)], tok0_v, sem_i)
    c1 = pltpu.async_copy(tok_hbm.at[pl.ds(base + HG, HG)], tok1_v, sem_i)
    c2 = pltpu.async_copy(slot_hbm.at[pl.ds(base, HG)], slot0_v, sem_i)
    c3 = pltpu.async_copy(slot_hbm.at[pl.ds(base + HG, HG)], slot1_v, sem_i)
    c0.wait()
    c1.wait()
    c2.wait()
    c3.wait()
    g0 = pltpu.async_copy(hs_hbm.at[tok0_v], rows0_v, sem_g0)
    g1 = pltpu.async_copy(hs_hbm.at[tok1_v], rows1_v, sem_g1)
    g0.wait()
    s0 = pltpu.async_copy(rows0_v, xs_hbm.at[slot0_v], sem_s)
    g1.wait()
    s1 = pltpu.async_copy(rows1_v, xs_hbm.at[slot1_v], sem_s)
    s0.wait()
    s1.wait()


HC = CB // 2


def _sc_combine_body(ys_hbm, sa_hbm, sb_hbm, wa_hbm, wb_hbm, out_hbm,
                     sa_v, sb_v, ya_v, yb_v, wa_v, wb_v,
                     sem_i, sem_a, sem_b, sem_o):
    wid = lax.axis_index("s") * 2 + lax.axis_index("c")
    base = wid * CB
    c0 = pltpu.async_copy(sa_hbm.at[pl.ds(base, CB)], sa_v, sem_i)
    c1 = pltpu.async_copy(sb_hbm.at[pl.ds(base, CB)], sb_v, sem_i)
    c2 = pltpu.async_copy(wa_hbm.at[pl.ds(base * LANES, CB * LANES)], wa_v, sem_i)
    c3 = pltpu.async_copy(wb_hbm.at[pl.ds(base * LANES, CB * LANES)], wb_v, sem_i)
    c0.wait()
    c1.wait()
    c2.wait()
    c3.wait()
    ga0 = pltpu.async_copy(ys_hbm.at[sa_v.at[pl.ds(0, HC)]],
                           ya_v.at[pl.ds(0, HC)], sem_a)
    gb0 = pltpu.async_copy(ys_hbm.at[sb_v.at[pl.ds(0, HC)]],
                           yb_v.at[pl.ds(0, HC)], sem_a)
    ga1 = pltpu.async_copy(ys_hbm.at[sa_v.at[pl.ds(HC, HC)]],
                           ya_v.at[pl.ds(HC, HC)], sem_b)
    gb1 = pltpu.async_copy(ys_hbm.at[sb_v.at[pl.ds(HC, HC)]],
                           yb_v.at[pl.ds(HC, HC)], sem_b)

    def row_body(r, carry):
        wa = wa_v[pl.ds(r * LANES, LANES)]
        wb = wb_v[pl.ds(r * LANES, LANES)]
        for c in range(DC):
            ya = ya_v[r, pl.ds(c * LANES, LANES)]
            yb = yb_v[r, pl.ds(c * LANES, LANES)]
            ya_v[r, pl.ds(c * LANES, LANES)] = wa * ya + wb * yb
        return carry

    ga0.wait()
    gb0.wait()
    lax.fori_loop(0, HC, row_body, 0)
    o0 = pltpu.async_copy(ya_v.at[pl.ds(0, HC)],
                          out_hbm.at[pl.ds(base, HC)], sem_o)
    ga1.wait()
    gb1.wait()
    lax.fori_loop(HC, CB, row_body, 0)
    o1 = pltpu.async_copy(ya_v.at[pl.ds(HC, HC)],
                          out_hbm.at[pl.ds(base + HC, HC)], sem_o)
    o0.wait()
    o1.wait()


def kernel(hidden_states, W_gate, W_gate_up, W_down, num_global_tokens,
           max_num_tokens_per_gpu):
    hs = hidden_states
    topk_w, slots, te2, tx2, ty2 = pl.pallas_call(
        _router_kernel,
        out_shape=(
            jax.ShapeDtypeStruct((T, K), jnp.float32),
            jax.ShapeDtypeStruct((T, K), jnp.int32),
            jax.ShapeDtypeStruct((NT_PAD, 1), jnp.int32),
            jax.ShapeDtypeStruct((NT_PAD, 1), jnp.int32),
            jax.ShapeDtypeStruct((NT_PAD, 1), jnp.int32),
        ),
    )(hs, W_gate)
    tile_expert = te2.reshape(-1)
    tile_x = tx2.reshape(-1)
    tile_y = ty2.reshape(-1)
    t_flat = jnp.repeat(jnp.arange(T, dtype=jnp.int32), K)
    slot = slots.reshape(-1)

    sc_mesh = plsc.VectorSubcoreMesh(core_axis_name="c", subcore_axis_name="s")
    sc_dispatch = pl.kernel(
        _sc_dispatch_body,
        mesh=sc_mesh,
        out_type=jax.ShapeDtypeStruct((NG, D), jnp.float32),
        scratch_types=[
            pltpu.VMEM((HG,), jnp.int32),
            pltpu.VMEM((HG,), jnp.int32),
            pltpu.VMEM((HG,), jnp.int32),
            pltpu.VMEM((HG,), jnp.int32),
            pltpu.VMEM((HG, D), jnp.float32),
            pltpu.VMEM((HG, D), jnp.float32),
            pltpu.SemaphoreType.DMA,
            pltpu.SemaphoreType.DMA,
            pltpu.SemaphoreType.DMA,
            pltpu.SemaphoreType.DMA,
        ],
    )
    x_sorted = sc_dispatch(hs, t_flat, slot)

    grid_spec = pltpu.PrefetchScalarGridSpec(
        num_scalar_prefetch=3,
        grid=(N_TILES,),
        in_specs=[
            pl.BlockSpec((TM, D), lambda i, te, tx, ty: (tx[i], 0)),
            pl.BlockSpec((1, D, 2 * F), lambda i, te, tx, ty: (te[i], 0, 0)),
            pl.BlockSpec((1, F, D), lambda i, te, tx, ty: (te[i], 0, 0)),
        ],
        out_specs=pl.BlockSpec((TM, D), lambda i, te, tx, ty: (ty[i], 0)),
    )
    y_sorted = pl.pallas_call(
        _ffn_kernel,
        grid_spec=grid_spec,
        out_shape=jax.ShapeDtypeStruct(((N_TILES + 1) * TM, D), jnp.float32),
    )(tile_expert, tile_x, tile_y, x_sorted, W_gate_up, W_down)

    wa_b = jnp.broadcast_to(topk_w[:, 0:1], (T, LANES)).reshape(T * LANES)
    wb_b = jnp.broadcast_to(topk_w[:, 1:2], (T, LANES)).reshape(T * LANES)
    sc_combine = pl.kernel(
        _sc_combine_body,
        mesh=sc_mesh,
        out_type=jax.ShapeDtypeStruct((T, D), jnp.float32),
        scratch_types=[
            pltpu.VMEM((CB,), jnp.int32),
            pltpu.VMEM((CB,), jnp.int32),
            pltpu.VMEM((CB, D), jnp.float32),
            pltpu.VMEM((CB, D), jnp.float32),
            pltpu.VMEM((CB * LANES,), jnp.float32),
            pltpu.VMEM((CB * LANES,), jnp.float32),
            pltpu.SemaphoreType.DMA,
            pltpu.SemaphoreType.DMA,
            pltpu.SemaphoreType.DMA,
            pltpu.SemaphoreType.DMA,
        ],
    )
    out = sc_combine(y_sorted, slots[:, 0], slots[:, 1], wa_b, wb_b)
    return out
